# baseline (speedup 1.0000x reference)
import jax
import jax.numpy as jnp
from jax.experimental import pallas as pl
from jax.experimental.pallas import tpu as pltpu

B, N, L, R, C = 16, 8192, 512, 5, 2


def _chowder_kernel(x_ref, w1_ref, b1_ref, Wa_ref, ba_ref, Wb_ref, bb_ref,
                    Wc_ref, bc_ref, out_ref, s_scr, cat_scr):
    b = pl.program_id(0)
    w = w1_ref[:].reshape(1, L)
    s = jax.lax.dot_general(w, x_ref[0], (((1,), (1,)), ((), ())),
                            preferred_element_type=jnp.float32)
    s_scr[pl.ds(b, 1), :] = s

    def take_extreme(v, gidx, sign):
        big = jnp.int32(2**30)
        m = (jnp.max(v, axis=1, keepdims=True) if sign > 0
             else jnp.min(v, axis=1, keepdims=True))
        fi = jnp.min(jnp.where(v == m, gidx, big), axis=1, keepdims=True)
        v2 = jnp.where(gidx == fi,
                       jnp.float32(-jnp.inf) if sign > 0
                       else jnp.float32(jnp.inf),
                       v)
        return m, v2

    def extremes(vals):
        gidx = jax.lax.broadcasted_iota(jnp.int32, vals.shape, 1)
        maxs = []
        v = vals
        for _ in range(R):
            m, v = take_extreme(v, gidx, +1)
            maxs.append(m)
        mins = []
        v = vals
        for _ in range(R):
            m, v = take_extreme(v, gidx, -1)
            mins.append(m)
        return jnp.concatenate(mins + maxs, axis=1)

    H = B // 2

    @pl.when(b == H)
    def _finish_low():
        cat_scr[0:H, :] = extremes(s_scr[0:H, :] + b1_ref[0])

    @pl.when(b == B - 1)
    def _finish():
        cat_scr[H:B, :] = extremes(s_scr[H:B, :] + b1_ref[0])
        cat = cat_scr[...]
        h = jnp.dot(cat, Wa_ref[:].T,
                    preferred_element_type=jnp.float32) + ba_ref[:]
        h = jnp.dot(h, Wb_ref[:].T,
                    preferred_element_type=jnp.float32) + bb_ref[:]
        o = jnp.dot(h, Wc_ref[:].T,
                    preferred_element_type=jnp.float32) + bc_ref[:]
        out_ref[...] = o[:, None, :]


@jax.jit
def _chowder(x, w1, b1, Wa, ba, Wb, bb, Wc, bc):
    out = pl.pallas_call(
        _chowder_kernel,
        grid=(B,),
        in_specs=[
            pl.BlockSpec((1, N, L), lambda b: (b, 0, 0)),
            pl.BlockSpec((L,), lambda b: (0,)),
            pl.BlockSpec((1,), lambda b: (0,)),
            pl.BlockSpec((200, 2 * R), lambda b: (0, 0)),
            pl.BlockSpec((200,), lambda b: (0,)),
            pl.BlockSpec((100, 200), lambda b: (0, 0)),
            pl.BlockSpec((100,), lambda b: (0,)),
            pl.BlockSpec((C, 100), lambda b: (0, 0)),
            pl.BlockSpec((C,), lambda b: (0,)),
        ],
        out_specs=pl.BlockSpec((B, 1, C), lambda b: (0, 0, 0)),
        out_shape=jax.ShapeDtypeStruct((B, 1, C), jnp.float32),
        scratch_shapes=[pltpu.VMEM((B, N), jnp.float32),
                        pltpu.VMEM((B, 2 * R), jnp.float32)],
        compiler_params=pltpu.CompilerParams(
            dimension_semantics=("arbitrary",),
        ),
    )(x, w1, b1, Wa, ba, Wb, bb, Wc, bc)
    return out


def kernel(x, w1, b1, Wa, ba, Wb, bb, Wc, bc):
    out = _chowder(x.astype(jnp.float32), w1, b1, Wa, ba, Wb, bb, Wc, bc)
    return (out, None)


# speedup vs baseline: 1.0043x; 1.0043x over previous
import jax
import jax.numpy as jnp
from jax.experimental import pallas as pl
from jax.experimental.pallas import tpu as pltpu

B, N, L, R, C = 16, 8192, 512, 5, 2


def _chowder_kernel(x_ref, w1_ref, b1_ref, Wa_ref, ba_ref, Wb_ref, bb_ref,
                    Wc_ref, bc_ref, out_ref, s_scr):
    b = pl.program_id(0)
    w = w1_ref[:].reshape(1, L)
    s = jax.lax.dot_general(w, x_ref[0], (((1,), (1,)), ((), ())),
                            preferred_element_type=jnp.float32)
    s_scr[pl.ds(b, 1), :] = s

    @pl.when(b == B - 1)
    def _finish():
        vals = s_scr[...] + b1_ref[0]
        gidx = jax.lax.broadcasted_iota(jnp.int32, (B, N), 1)
        big = jnp.int32(2**30)

        def take_extreme(v, sign):
            m = (jnp.max(v, axis=1, keepdims=True) if sign > 0
                 else jnp.min(v, axis=1, keepdims=True))
            fi = jnp.min(jnp.where(v == m, gidx, big), axis=1, keepdims=True)
            v2 = jnp.where(gidx == fi,
                           jnp.float32(-jnp.inf) if sign > 0
                           else jnp.float32(jnp.inf),
                           v)
            return m, v2

        maxs = []
        v = vals
        for _ in range(R):
            m, v = take_extreme(v, +1)
            maxs.append(m)
        mins = []
        v = vals
        for _ in range(R):
            m, v = take_extreme(v, -1)
            mins.append(m)

        cat = jnp.concatenate(mins + maxs, axis=1)
        h = jnp.dot(cat, Wa_ref[:].T,
                    preferred_element_type=jnp.float32) + ba_ref[:]
        h = jnp.dot(h, Wb_ref[:].T,
                    preferred_element_type=jnp.float32) + bb_ref[:]
        o = jnp.dot(h, Wc_ref[:].T,
                    preferred_element_type=jnp.float32) + bc_ref[:]
        out_ref[...] = o[:, None, :]


@jax.jit
def _chowder(x, w1, b1, Wa, ba, Wb, bb, Wc, bc):
    out = pl.pallas_call(
        _chowder_kernel,
        grid=(B,),
        in_specs=[
            pl.BlockSpec((1, N, L), lambda b: (b, 0, 0)),
            pl.BlockSpec((L,), lambda b: (0,)),
            pl.BlockSpec((1,), lambda b: (0,)),
            pl.BlockSpec((200, 2 * R), lambda b: (0, 0)),
            pl.BlockSpec((200,), lambda b: (0,)),
            pl.BlockSpec((100, 200), lambda b: (0, 0)),
            pl.BlockSpec((100,), lambda b: (0,)),
            pl.BlockSpec((C, 100), lambda b: (0, 0)),
            pl.BlockSpec((C,), lambda b: (0,)),
        ],
        out_specs=pl.BlockSpec((B, 1, C), lambda b: (0, 0, 0)),
        out_shape=jax.ShapeDtypeStruct((B, 1, C), jnp.float32),
        scratch_shapes=[pltpu.VMEM((B, N), jnp.float32)],
        compiler_params=pltpu.CompilerParams(
            dimension_semantics=("arbitrary",),
        ),
    )(x, w1, b1, Wa, ba, Wb, bb, Wc, bc)
    return out


def kernel(x, w1, b1, Wa, ba, Wb, bb, Wc, bc):
    out = _chowder(x.astype(jnp.float32), w1, b1, Wa, ba, Wb, bb, Wc, bc)
    return (out, None)
